# Initial kernel scaffold; baseline (speedup 1.0000x reference)
import functools

import jax
import jax.numpy as jnp
from jax import lax
from jax.experimental import pallas as pl
from jax.experimental.pallas import tpu as pltpu
from jax.experimental.pallas import tpu_sc as plsc

H = 8
VOCAB = 100000
DIM = 64
B = 1024
T = 50
NTOK = B * T
NC = 2
NS = 16
NW = NC * NS
TPW = NTOK // NW
CK = 80
NCH = TPW // CK
GPC = CK // 16
NG = TPW // 16

_mesh = plsc.VectorSubcoreMesh(core_axis_name="c", subcore_axis_name="s")


@functools.partial(
    pl.kernel,
    mesh=_mesh,
    out_type=[
        jax.ShapeDtypeStruct((H * NTOK, DIM), jnp.float32),
        jax.ShapeDtypeStruct((NTOK // 16, 16), jnp.int32),
    ],
    scratch_types=[
        pltpu.VMEM((TPW, H), jnp.int32),
        pltpu.VMEM((NCH, CK), jnp.int32),
        pltpu.VMEM((NG, 16), jnp.int32),
        pltpu.VMEM((CK, DIM), jnp.float32),
        pltpu.SemaphoreType.DMA,
    ],
)
def _emb_kernel(codecs_hbm, w_hbm, emb_hbm, mask_hbm,
                codecs_v, idx_v, macc_v, rows_v, sem):
    wid = lax.axis_index("s") * NC + lax.axis_index("c")
    tok0 = wid * TPW
    pltpu.sync_copy(codecs_hbm.at[pl.ds(tok0, TPW)], codecs_v)
    lanes = lax.iota(jnp.int32, 16)

    for h in range(H):
        base = h * VOCAB
        cols = jnp.full((16,), h, jnp.int32)

        def extract(c, _, h=h, base=base, cols=cols):
            for k in range(GPC):
                g = c * GPC + k
                rows = g * 16 + lanes
                vals = plsc.load_gather(codecs_v, [rows, cols])
                idx_v[c, pl.ds(k * 16, 16)] = vals + base
                eq = jnp.where(vals == 0, 1, 0).astype(jnp.int32)
                if h == 0:
                    macc_v[g, :] = eq
                else:
                    macc_v[g, :] = macc_v[g, :] & eq
            return 0

        lax.fori_loop(0, NCH, extract, 0)

        def fetch(c, _, h=h, base=base):
            pltpu.async_copy(w_hbm.at[idx_v.at[c]], rows_v, sem).wait()
            for k in range(GPC):
                v16 = idx_v[c, pl.ds(k * 16, 16)]
                nz = jnp.sum(jnp.where(v16 == base, 1, 0).astype(jnp.int32))

                @pl.when(nz > 0)
                def _fix(c=c, k=k, base=base):
                    def zero_row(r, _):
                        s = idx_v[c, k * 16 + r]

                        @pl.when(s == base)
                        def _z():
                            for q in range(DIM // 16):
                                rows_v[k * 16 + r, pl.ds(q * 16, 16)] = (
                                    jnp.zeros((16,), jnp.float32))
                        return 0

                    lax.fori_loop(0, 16, zero_row, 0)

            pltpu.sync_copy(
                rows_v,
                emb_hbm.at[pl.ds(h * NTOK + tok0 + c * CK, CK)])
            return 0

        lax.fori_loop(0, NCH, fetch, 0)

    pltpu.sync_copy(macc_v, mask_hbm.at[pl.ds(wid * NG, NG)])


def kernel(codecs, W):
    codecs_flat = codecs.reshape(NTOK, H)
    w_flat = W.reshape(H * VOCAB, DIM)
    emb, mask_i32 = _emb_kernel(codecs_flat, w_flat)
    emb = emb.reshape(H, B, T, DIM)
    mask = mask_i32.reshape(B, T).astype(bool)
    return (emb, mask)


# speedup vs baseline: 1.4118x; 1.4118x over previous
import functools

import jax
import jax.numpy as jnp
from jax import lax
from jax.experimental import pallas as pl
from jax.experimental.pallas import tpu as pltpu
from jax.experimental.pallas import tpu_sc as plsc

H = 8
VOCAB = 100000
DIM = 64
B = 1024
T = 50
NTOK = B * T
NC = 2
NS = 16
NW = NC * NS
TPW = NTOK // NW
CK = 80
NCH = TPW // CK
GPC = CK // 16

_mesh = plsc.VectorSubcoreMesh(core_axis_name="c", subcore_axis_name="s")


@functools.partial(
    pl.kernel,
    mesh=_mesh,
    compiler_params=pltpu.CompilerParams(use_tc_tiling_on_sc=False),
    out_type=[
        jax.ShapeDtypeStruct((H * NTOK, DIM), jnp.float32),
        jax.ShapeDtypeStruct((NTOK,), jnp.int32),
    ],
    scratch_types=[
        pltpu.VMEM((TPW,), jnp.int32),
        pltpu.VMEM((TPW,), jnp.int32),
        pltpu.VMEM((CK, DIM), jnp.float32),
        pltpu.SemaphoreType.DMA,
    ],
)
def _emb_kernel(codecs_hbm, w_hbm, emb_hbm, mask_hbm,
                idx_v, macc_v, rows_v, sem):
    wid = lax.axis_index("s") * NC + lax.axis_index("c")
    tok0 = wid * TPW

    for h in range(H):
        base = h * VOCAB
        pltpu.sync_copy(codecs_hbm.at[pl.ds(h * NTOK + tok0, TPW)], idx_v)

        def chunk(c, _, h=h, base=base):
            orv = None
            for k in range(GPC):
                off = c * CK + k * 16
                v = idx_v[pl.ds(off, 16)]
                eq = jnp.where(v == 0, 1, 0).astype(jnp.int32)
                orv = eq if orv is None else (orv | eq)
                idx_v[pl.ds(off, 16)] = v + base
                if h == 0:
                    macc_v[pl.ds(off, 16)] = eq
                else:
                    macc_v[pl.ds(off, 16)] = macc_v[pl.ds(off, 16)] & eq
            any_zero = orv[0]
            for l in range(1, 16):
                any_zero = any_zero | orv[l]

            pltpu.async_copy(
                w_hbm.at[idx_v.at[pl.ds(c * CK, CK)]], rows_v, sem).wait()

            @pl.when(any_zero > 0)
            def _fix(c=c, base=base):
                zeros = jnp.zeros((16,), jnp.float32)
                for k in range(GPC):
                    v16 = idx_v[pl.ds(c * CK + k * 16, 16)]
                    for l in range(16):
                        @pl.when(v16[l] == base)
                        def _z(k=k, l=l, zeros=zeros):
                            for q in range(DIM // 16):
                                rows_v[k * 16 + l, pl.ds(q * 16, 16)] = zeros

            pltpu.sync_copy(
                rows_v,
                emb_hbm.at[pl.ds(h * NTOK + tok0 + c * CK, CK)])
            return 0

        lax.fori_loop(0, NCH, chunk, 0)

    pltpu.sync_copy(macc_v, mask_hbm.at[pl.ds(tok0, TPW)])


def kernel(codecs, W):
    codecs_t = jnp.transpose(codecs.reshape(NTOK, H)).reshape(H * NTOK)
    w_flat = W.reshape(H * VOCAB, DIM)
    emb, mask_i32 = _emb_kernel(codecs_t, w_flat)
    emb = emb.reshape(H, B, T, DIM)
    mask = mask_i32.reshape(B, T).astype(bool)
    return (emb, mask)


# speedup vs baseline: 1.6199x; 1.1474x over previous
import functools

import jax
import jax.numpy as jnp
from jax import lax
from jax.experimental import pallas as pl
from jax.experimental.pallas import tpu as pltpu
from jax.experimental.pallas import tpu_sc as plsc

H = 8
VOCAB = 100000
DIM = 64
B = 1024
T = 50
NTOK = B * T
NC = 2
NS = 16
NW = NC * NS
TPW = NTOK // NW
CK = 80
HCK = TPW // 2
NCH = HCK // CK
GP16 = HCK // 16

_mesh = plsc.VectorSubcoreMesh(core_axis_name="c", subcore_axis_name="s")


@functools.partial(
    pl.kernel,
    mesh=_mesh,
    compiler_params=pltpu.CompilerParams(use_tc_tiling_on_sc=False),
    out_type=[
        jax.ShapeDtypeStruct((H * NTOK, DIM), jnp.float32),
        jax.ShapeDtypeStruct((NTOK,), jnp.int32),
    ],
    scratch_types=[
        pltpu.VMEM((TPW,), jnp.int32),
        pltpu.VMEM((TPW,), jnp.int32),
        pltpu.VMEM((HCK, DIM), jnp.float32),
        pltpu.VMEM((HCK, DIM), jnp.float32),
        pltpu.SemaphoreType.DMA,
        pltpu.SemaphoreType.DMA,
        pltpu.SemaphoreType.DMA,
    ],
)
def _emb_kernel(codecs_hbm, w_hbm, emb_hbm, mask_hbm,
                idx_v, macc_v, rows0_v, rows1_v, gsem, osem0, osem1):
    wid = lax.axis_index("s") * NC + lax.axis_index("c")
    tok0 = wid * TPW
    rows_bufs = (rows0_v, rows1_v)
    osems = (osem0, osem1)

    def init_mask(g, _):
        macc_v[pl.ds(g * 16, 16)] = jnp.full((16,), 1, jnp.int32)
        return 0

    lax.fori_loop(0, TPW // 16, init_mask, 0)

    def head(h, _):
        base = h * VOCAB
        pltpu.sync_copy(codecs_hbm.at[pl.ds(h * NTOK + tok0, TPW)], idx_v)

        for p in (0, 1):
            rows_v = rows_bufs[p]
            osem = osems[p]
            h0 = p * HCK

            def bias(g, orv, h0=h0):
                off = h0 + g * 16
                v = idx_v[pl.ds(off, 16)]
                eq = jnp.where(v == 0, 1, 0).astype(jnp.int32)
                idx_v[pl.ds(off, 16)] = v + base
                macc_v[pl.ds(off, 16)] = macc_v[pl.ds(off, 16)] & eq
                return orv | eq

            orv = lax.fori_loop(0, GP16, bias, jnp.zeros((16,), jnp.int32))
            any_zero = orv[0]
            for l in range(1, 16):
                any_zero = any_zero | orv[l]

            @pl.when(h > 0)
            def _drain_prev(rows_v=rows_v, osem=osem):
                pltpu.make_async_copy(
                    rows_v, emb_hbm.at[pl.ds(0, HCK)], osem).wait()

            descs = []
            for j in range(NCH):
                descs.append(pltpu.async_copy(
                    w_hbm.at[idx_v.at[pl.ds(h0 + j * CK, CK)]],
                    rows_v.at[pl.ds(j * CK, CK)], gsem))
            for d in descs:
                d.wait()

            @pl.when(any_zero > 0)
            def _fix(rows_v=rows_v, h0=h0, base=base):
                zeros = jnp.zeros((16,), jnp.float32)

                def fix_group(g, _):
                    v16 = idx_v[pl.ds(h0 + g * 16, 16)]
                    for l in range(16):
                        @pl.when(v16[l] == base)
                        def _z(l=l):
                            for q in range(DIM // 16):
                                rows_v[g * 16 + l, pl.ds(q * 16, 16)] = zeros
                    return 0

                lax.fori_loop(0, GP16, fix_group, 0)

            pltpu.async_copy(
                rows_v,
                emb_hbm.at[pl.ds(h * NTOK + tok0 + h0, HCK)], osem)
        return 0

    lax.fori_loop(0, H, head, 0)

    for p in (0, 1):
        pltpu.make_async_copy(
            rows_bufs[p], emb_hbm.at[pl.ds(0, HCK)], osems[p]).wait()

    pltpu.sync_copy(macc_v, mask_hbm.at[pl.ds(tok0, TPW)])


def kernel(codecs, W):
    codecs_t = jnp.transpose(codecs.reshape(NTOK, H)).reshape(H * NTOK)
    w_flat = W.reshape(H * VOCAB, DIM)
    emb, mask_i32 = _emb_kernel(codecs_t, w_flat)
    emb = emb.reshape(H, B, T, DIM)
    mask = mask_i32.reshape(B, T).astype(bool)
    return (emb, mask)
